# baseline (speedup 1.0000x reference)
import functools

import jax
import jax.numpy as jnp
from jax import lax
from jax.experimental import pallas as pl
from jax.experimental.pallas import tpu as pltpu
from jax.experimental.pallas import tpu_sc as plsc

NUM_FIELDS = 26
VOCAB = 100000
EMBED_DIM = 32
DENSE_DIM = 13
BATCH = 16384
CONCAT = NUM_FIELDS * EMBED_DIM
CPAD = 896
NBANDS = BATCH // 8

NC, NS = 2, 16
NW = NC * NS
BANDS_PER_W = NBANDS // NW
CHUNK_BANDS = 8
N_CHUNKS = BANDS_PER_W // CHUNK_BANDS
IDX_MINOR = 128


def _sc_gather(sidx, tbl, n_tiles):
    rows_per_w = BANDS_PER_W * n_tiles * 32
    rows_per_chunk = rows_per_w // N_CHUNKS
    idx_rows_per_chunk = rows_per_chunk // IDX_MINOR
    idx_rows_per_w = rows_per_w // IDX_MINOR
    total_rows = NBANDS * n_tiles * 32
    mesh = plsc.VectorSubcoreMesh(core_axis_name="c", subcore_axis_name="s")

    @functools.partial(
        pl.kernel,
        mesh=mesh,
        out_type=jax.ShapeDtypeStruct((total_rows, EMBED_DIM), jnp.float32),
        compiler_params=pltpu.CompilerParams(use_tc_tiling_on_sc=False),
        scratch_types=[
            pltpu.VMEM((idx_rows_per_w, IDX_MINOR), jnp.int32),
            pltpu.VMEM((rows_per_chunk, EMBED_DIM), jnp.float32),
            pltpu.SemaphoreType.DMA,
        ],
    )
    def gather_kernel(sidx_hbm, tbl_hbm, out_hbm, idx_v, rows_v, sem):
        wid = lax.axis_index("s") * NC + lax.axis_index("c")
        pltpu.sync_copy(
            sidx_hbm.at[pl.ds(wid * idx_rows_per_w, idx_rows_per_w)], idx_v)

        def chunk_body(c, carry):
            cps = [
                pltpu.async_copy(
                    tbl_hbm.at[idx_v.at[c * idx_rows_per_chunk + j]],
                    rows_v.at[pl.ds(j * IDX_MINOR, IDX_MINOR)],
                    sem,
                )
                for j in range(idx_rows_per_chunk)
            ]
            for cp in cps:
                cp.wait()
            pltpu.sync_copy(
                rows_v,
                out_hbm.at[pl.ds(wid * rows_per_w + c * rows_per_chunk,
                                 rows_per_chunk)],
            )
            return carry

        lax.fori_loop(0, N_CHUNKS, chunk_body, 0)

    return gather_kernel(sidx, tbl)


def _dense_body(dx_ref, ga_ref, gb_ref, w1_ref, b1_ref, w2_ref, b2_ref,
                w3p_ref, w3b_ref, b3_ref, w4_ref, b4_ref, out_ref):
    h = jnp.maximum(
        jnp.dot(dx_ref[...], w1_ref[...], preferred_element_type=jnp.float32)
        + b1_ref[...], 0.0)
    d = jnp.dot(h, w2_ref[...], preferred_element_type=jnp.float32) + b2_ref[...]
    t = (jnp.dot(d, w3b_ref[...], preferred_element_type=jnp.float32)
         + b3_ref[...])
    for j in range(7):
        mj = (ga_ref[:, j] if j < 4 else gb_ref[:, j - 4]).reshape(_BT, 128)
        t = t + jnp.dot(mj, w3p_ref[j * 128:(j + 1) * 128, :],
                        preferred_element_type=jnp.float32)
    h2 = jnp.maximum(t, 0.0)
    z = jnp.dot(h2, w4_ref[...], preferred_element_type=jnp.float32) + b4_ref[...]
    out_ref[...] = 1.0 / (1.0 + jnp.exp(-z))


_BT = 2048


def _dense_forward(dense_x, ga, gb, W1, b1, W2, b2, W3p, W3b, b3, W4, b4):
    fixed = lambda t: (0, 0)
    tiled = lambda t: (t, 0)
    return pl.pallas_call(
        _dense_body,
        grid=(BATCH // _BT,),
        in_specs=[
            pl.BlockSpec((_BT, DENSE_DIM), tiled),
            pl.BlockSpec((_BT // 8, 4, 8, 128), lambda t: (t, 0, 0, 0)),
            pl.BlockSpec((_BT // 8, 3, 8, 128), lambda t: (t, 0, 0, 0)),
            pl.BlockSpec((DENSE_DIM, 8), fixed),
            pl.BlockSpec((1, 8), fixed),
            pl.BlockSpec((8, EMBED_DIM), fixed),
            pl.BlockSpec((1, EMBED_DIM), fixed),
            pl.BlockSpec((CPAD, 16), fixed),
            pl.BlockSpec((EMBED_DIM, 16), fixed),
            pl.BlockSpec((1, 16), fixed),
            pl.BlockSpec((16, 1), fixed),
            pl.BlockSpec((1, 1), fixed),
        ],
        out_specs=pl.BlockSpec((_BT, 1), tiled),
        out_shape=jax.ShapeDtypeStruct((BATCH, 1), jnp.float32),
    )(dense_x, ga, gb, W1, b1, W2, b2, W3p, W3b, b3, W4, b4)


def kernel(dense_x, sparse_x, emb_tables, W1, b1, W2, b2, W3, b3, W4, b4):
    offs = (jnp.arange(NUM_FIELDS, dtype=jnp.int32) * VOCAB)[None, :]
    fidx = sparse_x + offs

    ka = fidx[:, :16].reshape(NBANDS, 8, 4, 4).transpose(0, 2, 1, 3)
    ka = ka.reshape(NBANDS * 4 * 32 // 128, 128)
    fb = fidx[:, 16:] - 16 * VOCAB
    kb = jnp.concatenate([fb, fb[:, :2]], axis=1)
    kb = kb.reshape(NBANDS, 8, 3, 4).transpose(0, 2, 1, 3)
    kb = kb.reshape(NBANDS * 3 * 32 // 128, 128)

    tbl_a = emb_tables[:16].reshape(16 * VOCAB, EMBED_DIM)
    tbl_b = emb_tables[16:].reshape(10 * VOCAB, EMBED_DIM)

    ga = _sc_gather(ka, tbl_a, 4).reshape(NBANDS, 4, 8, 128)
    gb = _sc_gather(kb, tbl_b, 3).reshape(NBANDS, 3, 8, 128)

    w3a = W3[:CONCAT].reshape(NUM_FIELDS, EMBED_DIM, 16)
    w3p = jnp.zeros((7, 4, EMBED_DIM, 16), jnp.float32)
    w3p = w3p.at[:6].set(w3a[:24].reshape(6, 4, EMBED_DIM, 16))
    w3p = w3p.at[6, :2].set(w3a[24:26])
    w3p = w3p.reshape(CPAD, 16)
    return _dense_forward(
        dense_x, ga, gb, W1, b1.reshape(1, 8), W2, b2.reshape(1, EMBED_DIM),
        w3p, W3[CONCAT:], b3.reshape(1, 16), W4, b4.reshape(1, 1))
